# baseline (speedup 1.0000x reference)
import math

import jax
import jax.numpy as jnp
from jax import lax
from jax.experimental import pallas as pl
from jax.experimental.pallas import tpu as pltpu
from jax.experimental.pallas import tpu_sc as plsc

SMOOTHING = 0.1
N_CLASSES = 100000
CONFIDENCE = 1.0 - SMOOTHING
SMOOTHING_VALUE = SMOOTHING / (N_CLASSES - 1)

ROW_BLOCK = 32

_SC_CORES = 2
_SC_SUBCORES = 16
_SC_WORKERS = _SC_CORES * _SC_SUBCORES

def _gather_body(table_hbm, idx_hbm, out_hbm, idx_v, vals_v, sem):
    b_per_w = idx_v.shape[0]
    wid = lax.axis_index("s") * _SC_CORES + lax.axis_index("c")
    base = wid * b_per_w
    pltpu.sync_copy(idx_hbm.at[pl.ds(base, b_per_w)], idx_v)
    pltpu.async_copy(table_hbm.at[idx_v], vals_v, sem).wait()
    pltpu.sync_copy(vals_v, out_hbm.at[pl.ds(base, b_per_w)])


def _sc_gather(table, fidx):
    B = fidx.shape[0]
    b_per_w = B // _SC_WORKERS
    mesh = plsc.VectorSubcoreMesh(core_axis_name="c", subcore_axis_name="s")
    return pl.kernel(
        _gather_body,
        out_type=jax.ShapeDtypeStruct((B,), jnp.float32),
        mesh=mesh,
        scratch_types=[
            pltpu.VMEM((b_per_w,), jnp.int32),
            pltpu.VMEM((b_per_w,), jnp.float32),
            pltpu.SemaphoreType.DMA,
        ],
    )(table, fidx)


def _loss_kernel(g_ref, x_ref, o_ref):
    i = pl.program_id(0)
    x = x_ref[...]
    g = g_ref[0]
    m = jnp.max(x, axis=1, keepdims=True)
    s = jnp.sum(jnp.exp(x - m), axis=1, keepdims=True)
    lse = m + jnp.log(s)
    row_sum = jnp.sum(x, axis=1, keepdims=True)
    partial = jnp.sum(
        lse
        - SMOOTHING_VALUE * row_sum
        - (CONFIDENCE - SMOOTHING_VALUE) * g
    ).reshape(1, 1)

    @pl.when(i == 0)
    def _init():
        B = ROW_BLOCK * pl.num_programs(0)
        const = B * (
            (N_CLASSES - 1) * SMOOTHING_VALUE * math.log(SMOOTHING_VALUE)
            + CONFIDENCE * math.log(CONFIDENCE)
        )
        o_ref[...] = jnp.full((1, 1), const, dtype=jnp.float32)

    o_ref[...] += partial


@jax.jit
def kernel(output, target):
    B, N = output.shape
    t32 = target.astype(jnp.int32)
    flat = output.reshape(B * N)
    fidx = jnp.arange(B, dtype=jnp.int32) * N + t32
    g = _sc_gather(flat, fidx)
    n_blocks = B // ROW_BLOCK
    g3 = g.reshape(n_blocks, ROW_BLOCK, 1)
    out = pl.pallas_call(
        _loss_kernel,
        grid=(n_blocks,),
        in_specs=[
            pl.BlockSpec((1, ROW_BLOCK, 1), lambda i: (i, 0, 0)),
            pl.BlockSpec((ROW_BLOCK, N), lambda i: (i, 0)),
        ],
        out_specs=pl.BlockSpec((1, 1), lambda i: (0, 0)),
        out_shape=jax.ShapeDtypeStruct((1, 1), jnp.float32),
    )(g3, output)
    return out[0, 0]


# speedup vs baseline: 1.0028x; 1.0028x over previous
import math

import jax
import jax.numpy as jnp
from jax import lax
from jax.experimental import pallas as pl
from jax.experimental.pallas import tpu as pltpu
from jax.experimental.pallas import tpu_sc as plsc

SMOOTHING = 0.1
N_CLASSES = 100000
CONFIDENCE = 1.0 - SMOOTHING
SMOOTHING_VALUE = SMOOTHING / (N_CLASSES - 1)

ROW_BLOCK = 32

_SC_CORES = 2
_SC_SUBCORES = 16
_SC_WORKERS = _SC_CORES * _SC_SUBCORES

def _gather_body(table_hbm, idx_hbm, out_hbm, idx_v, vals_v, sem):
    b_per_w = idx_v.shape[0]
    wid = lax.axis_index("s") * _SC_CORES + lax.axis_index("c")
    base = wid * b_per_w
    pltpu.sync_copy(idx_hbm.at[pl.ds(base, b_per_w)], idx_v)
    pltpu.async_copy(table_hbm.at[idx_v], vals_v, sem).wait()
    pltpu.sync_copy(vals_v, out_hbm.at[pl.ds(base, b_per_w)])


def _sc_gather(table, fidx):
    B = fidx.shape[0]
    b_per_w = B // _SC_WORKERS
    mesh = plsc.VectorSubcoreMesh(core_axis_name="c", subcore_axis_name="s")
    return pl.kernel(
        _gather_body,
        out_type=jax.ShapeDtypeStruct((B,), jnp.float32),
        mesh=mesh,
        scratch_types=[
            pltpu.VMEM((b_per_w,), jnp.int32),
            pltpu.VMEM((b_per_w,), jnp.float32),
            pltpu.SemaphoreType.DMA,
        ],
    )(table, fidx)


def _dense_kernel(x_ref, o_ref):
    i = pl.program_id(0)
    x = x_ref[...]
    m = jnp.max(x, axis=1, keepdims=True)
    s = jnp.sum(jnp.exp(x - m), axis=1, keepdims=True)
    lse = m + jnp.log(s)
    row_sum = jnp.sum(x, axis=1, keepdims=True)
    partial = jnp.sum(lse - SMOOTHING_VALUE * row_sum).reshape(1, 1)

    @pl.when(i == 0)
    def _init():
        B = ROW_BLOCK * pl.num_programs(0)
        const = B * (
            (N_CLASSES - 1) * SMOOTHING_VALUE * math.log(SMOOTHING_VALUE)
            + CONFIDENCE * math.log(CONFIDENCE)
        )
        o_ref[...] = jnp.full((1, 1), const, dtype=jnp.float32)

    o_ref[...] += partial


def _combine_kernel(p_ref, g_ref, o_ref):
    gsum = jnp.sum(g_ref[...])
    o_ref[...] = p_ref[...] - (CONFIDENCE - SMOOTHING_VALUE) * gsum


@jax.jit
def kernel(output, target):
    B, N = output.shape
    t32 = target.astype(jnp.int32)
    flat = output.reshape(B * N)
    fidx = jnp.arange(B, dtype=jnp.int32) * N + t32
    g = _sc_gather(flat, fidx)
    n_blocks = B // ROW_BLOCK
    part = pl.pallas_call(
        _dense_kernel,
        grid=(n_blocks,),
        in_specs=[pl.BlockSpec((ROW_BLOCK, N), lambda i: (i, 0))],
        out_specs=pl.BlockSpec((1, 1), lambda i: (0, 0)),
        out_shape=jax.ShapeDtypeStruct((1, 1), jnp.float32),
    )(output)
    out = pl.pallas_call(
        _combine_kernel,
        out_shape=jax.ShapeDtypeStruct((1, 1), jnp.float32),
    )(part, g.reshape(8, B // 8))
    return out[0, 0]


# speedup vs baseline: 1.3668x; 1.3630x over previous
import math

import jax
import jax.numpy as jnp
from jax import lax
from jax.experimental import pallas as pl
from jax.experimental.pallas import tpu as pltpu
from jax.experimental.pallas import tpu_sc as plsc

SMOOTHING = 0.1
N_CLASSES = 100000
CONFIDENCE = 1.0 - SMOOTHING
SMOOTHING_VALUE = SMOOTHING / (N_CLASSES - 1)

ROW_BLOCK = 32

_SC_CORES = 2
_SC_SUBCORES = 16
_SC_WORKERS = _SC_CORES * _SC_SUBCORES

def _gather_body(x_hbm, idx_hbm, out_hbm, flat_hbm, row_v, idx_v, vals_v, sem):
    n = x_hbm.shape[1]
    b_per_w = idx_v.shape[0]
    wid = lax.axis_index("s") * _SC_CORES + lax.axis_index("c")
    base = wid * b_per_w
    for j in range(b_per_w):
        pltpu.sync_copy(x_hbm.at[base + j], row_v)
        pltpu.sync_copy(row_v, flat_hbm.at[pl.ds((base + j) * n, n)])
    pltpu.sync_copy(idx_hbm.at[pl.ds(base, b_per_w)], idx_v)
    pltpu.async_copy(flat_hbm.at[idx_v], vals_v, sem).wait()
    pltpu.sync_copy(vals_v, out_hbm.at[pl.ds(base, b_per_w)])


def _sc_gather(x, fidx):
    B, N = x.shape
    b_per_w = B // _SC_WORKERS
    mesh = plsc.VectorSubcoreMesh(core_axis_name="c", subcore_axis_name="s")
    return pl.kernel(
        _gather_body,
        out_type=jax.ShapeDtypeStruct((B,), jnp.float32),
        mesh=mesh,
        scratch_types=[
            pltpu.HBM((B * N,), jnp.float32),
            pltpu.VMEM((N,), jnp.float32),
            pltpu.VMEM((b_per_w,), jnp.int32),
            pltpu.VMEM((b_per_w,), jnp.float32),
            pltpu.SemaphoreType.DMA,
        ],
    )(x, fidx)


def _dense_kernel(x_ref, o_ref):
    i = pl.program_id(0)
    x = x_ref[...]
    m = jnp.max(x, axis=1, keepdims=True)
    s = jnp.sum(jnp.exp(x - m), axis=1, keepdims=True)
    lse = m + jnp.log(s)
    row_sum = jnp.sum(x, axis=1, keepdims=True)
    partial = jnp.sum(lse - SMOOTHING_VALUE * row_sum).reshape(1, 1)

    @pl.when(i == 0)
    def _init():
        B = ROW_BLOCK * pl.num_programs(0)
        const = B * (
            (N_CLASSES - 1) * SMOOTHING_VALUE * math.log(SMOOTHING_VALUE)
            + CONFIDENCE * math.log(CONFIDENCE)
        )
        o_ref[...] = jnp.full((1, 1), const, dtype=jnp.float32)

    o_ref[...] += partial


def _combine_kernel(p_ref, g_ref, o_ref):
    gsum = jnp.sum(g_ref[...])
    o_ref[...] = p_ref[...] - (CONFIDENCE - SMOOTHING_VALUE) * gsum


@jax.jit
def kernel(output, target):
    B, N = output.shape
    t32 = target.astype(jnp.int32)
    fidx = jnp.arange(B, dtype=jnp.int32) * N + t32
    g = _sc_gather(output, fidx)
    n_blocks = B // ROW_BLOCK
    part = pl.pallas_call(
        _dense_kernel,
        grid=(n_blocks,),
        in_specs=[pl.BlockSpec((ROW_BLOCK, N), lambda i: (i, 0))],
        out_specs=pl.BlockSpec((1, 1), lambda i: (0, 0)),
        out_shape=jax.ShapeDtypeStruct((1, 1), jnp.float32),
    )(output)
    out = pl.pallas_call(
        _combine_kernel,
        out_shape=jax.ShapeDtypeStruct((1, 1), jnp.float32),
    )(part, g.reshape(8, B // 8))
    return out[0, 0]
